# baseline (speedup 1.0000x reference)
import functools

import jax
import jax.numpy as jnp
from jax import lax
from jax.experimental import pallas as pl
from jax.experimental.pallas import tpu as pltpu
from jax.experimental.pallas import tpu_sc as plsc

N = 50000
E = 800000
HID = 64
NUM_GRAPHS = 512

NP = 51200
EP = 802816
NR = EP // 128
PAD_NODE = N

NW = 32
RW = NR // NW
BE = 2048
EG = EP // BE
BN = 2048
NG = NP // BN

_mesh = plsc.VectorSubcoreMesh(core_axis_name="c", subcore_axis_name="s")
_f32 = jnp.float32
_SCPARAMS = pltpu.CompilerParams(use_tc_tiling_on_sc=False)


_KR = 2
_NCH = RW // _KR


@functools.partial(
    pl.kernel,
    out_type=jax.ShapeDtypeStruct((EP, 128), _f32),
    mesh=_mesh,
    compiler_params=_SCPARAMS,
    scratch_types=[
        pltpu.VMEM((_KR, 128), jnp.int32),
        pltpu.VMEM((_KR, 128), jnp.int32),
        pltpu.VMEM((_KR, 128), jnp.int32),
        pltpu.VMEM((_KR, 128), jnp.int32),
        pltpu.VMEM((_KR * 128, 64), _f32),
        pltpu.VMEM((_KR * 128, 64), _f32),
        pltpu.VMEM((_KR * 128, 64), _f32),
        pltpu.VMEM((_KR * 128, 64), _f32),
        pltpu.SemaphoreType.DMA,
        pltpu.SemaphoreType.DMA,
    ],
)
def _sc_gather23(h, dst2, src2, out,
                 ia0, is0, ia1, is1, bd0, bs0, bd1, bs1, sem0, sem1):
    c = lax.axis_index("c")
    s = lax.axis_index("s")
    w = s * 2 + c
    wbase = w * RW

    def load_and_fire(chunk, ia, is_, bd, bs, sem):
        rbase = wbase + chunk * _KR
        pltpu.sync_copy(dst2.at[pl.ds(rbase, _KR), :], ia)
        pltpu.sync_copy(src2.at[pl.ds(rbase, _KR), :], is_)
        for j in range(_KR):
            sl = pl.ds(j * 128, 128)
            pltpu.async_copy(h.at[ia.at[j]], bd.at[sl, :], sem)
            pltpu.async_copy(h.at[is_.at[j]], bs.at[sl, :], sem)

    def drain_and_write(chunk, ia, is_, bd, bs, sem):
        for j in range(_KR):
            sl = pl.ds(j * 128, 128)
            pltpu.make_async_copy(h.at[ia.at[j]], bd.at[sl, :], sem).wait()
            pltpu.make_async_copy(h.at[is_.at[j]], bs.at[sl, :], sem).wait()
        ebase = (wbase + chunk * _KR) * 128
        pltpu.sync_copy(bd, out.at[pl.ds(ebase, _KR * 128), pl.ds(0, 64)])
        pltpu.sync_copy(bs, out.at[pl.ds(ebase, _KR * 128), pl.ds(64, 64)])

    load_and_fire(0, ia0, is0, bd0, bs0, sem0)

    def body(g, _):
        c0 = 2 * g
        load_and_fire(c0 + 1, ia1, is1, bd1, bs1, sem1)
        drain_and_write(c0, ia0, is0, bd0, bs0, sem0)

        @pl.when(c0 + 2 < _NCH)
        def _():
            load_and_fire(c0 + 2, ia0, is0, bd0, bs0, sem0)

        drain_and_write(c0 + 1, ia1, is1, bd1, bs1, sem1)
        return 0

    lax.fori_loop(0, _NCH // 2, body, 0)


_SKR = 7
_RPT = NR // 16
_SNCH = _RPT // _SKR


@functools.partial(
    pl.kernel,
    out_type=jax.ShapeDtypeStruct((NP, 64), _f32),
    mesh=_mesh,
    compiler_params=_SCPARAMS,
    scratch_types=[
        pltpu.VMEM((_SKR, 128), jnp.int32),
        pltpu.VMEM((_SKR, 128), jnp.int32),
        pltpu.VMEM((_SKR * 128, 16), _f32),
        pltpu.VMEM((_SKR * 128, 16), _f32),
        pltpu.VMEM_SHARED((NP, 16), _f32),
        pltpu.SemaphoreType.DMA,
        pltpu.SemaphoreType.DMA,
        pltpu.SemaphoreType.DMA,
    ],
)
def _sc_scatter23(m, dst2, h, accout,
                  ix0, ix1, mb0, mb1, acc, sem0, sem1, lsem):
    c = lax.axis_index("c")
    s = lax.axis_index("s")
    rows = pl.ds(s * 3200, 3200)

    for q in range(2):
        cols = pl.ds(c * 32 + q * 16, 16)

        pltpu.sync_copy(h.at[rows, cols], acc.at[rows, :])
        plsc.subcore_barrier()

        def load(chunk, ix, mb, cols=cols):
            rbase = s * _RPT + chunk * _SKR
            pltpu.sync_copy(dst2.at[pl.ds(rbase, _SKR), :], ix)
            return pltpu.async_copy(
                m.at[pl.ds(rbase * 128, _SKR * 128), cols], mb, lsem)

        def scat(ix, mb, sem):
            for j in range(_SKR):
                pltpu.async_copy(mb.at[pl.ds(j * 128, 128), :],
                                 acc.at[ix.at[j]], sem, add=True)

        def drain(ix, mb, sem):
            for j in range(_SKR):
                pltpu.make_async_copy(mb.at[pl.ds(j * 128, 128), :],
                                      acc.at[ix.at[j]], sem).wait()

        load(0, ix0, mb0).wait()

        def body(g, _):
            c0 = 2 * g
            ld = load(c0 + 1, ix1, mb1)
            scat(ix0, mb0, sem0)
            ld.wait()
            drain(ix0, mb0, sem0)

            @pl.when(c0 + 2 < _SNCH)
            def _():
                load(c0 + 2, ix0, mb0).wait()

            scat(ix1, mb1, sem1)
            drain(ix1, mb1, sem1)
            return 0

        lax.fori_loop(0, _SNCH // 2, body, 0)
        plsc.subcore_barrier()
        pltpu.sync_copy(acc.at[rows, :], accout.at[rows, cols])
        plsc.subcore_barrier()


_PREC = jax.lax.Precision.HIGHEST


def _dot(a, b, prec=_PREC):
    return jnp.dot(a, b, preferred_element_type=_f32, precision=prec)


def _sigmoid(x):
    return 1.0 / (1.0 + jnp.exp(-x))


def _softplus(x):
    return jnp.maximum(x, 0.0) + jnp.log1p(jnp.exp(-jnp.abs(x)))


def _nodeprep1_body(x8, w1d, w1s, pw, pb, t1, xp):
    xv = x8[...]
    z = jnp.zeros((xv.shape[0], 32), _f32)
    t1[...] = jnp.concatenate([_dot(xv, w1d[...]), _dot(xv, w1s[...]), z],
                              axis=1)
    xp[...] = _dot(xv, pw[...]) + pb[...]


def _edge1_body(gg, eav, we1, b1, pw8, mout):
    g = gg[...]
    ea = eav[...]
    a = g[:, 0:16] + g[:, 80:96] + _dot3(ea, we1[...]) + b1[...]
    m = _sigmoid(a[:, 0:8]) * _softplus(a[:, 8:16])
    m1 = _dot(m, pw8[...])
    mout[...] = jnp.concatenate([m1, jnp.zeros_like(m1)], axis=1)


def _lnact_body(a, g, b, hout):
    h = a[...]
    mu = jnp.mean(h, axis=1, keepdims=True)
    var = jnp.mean((h - mu) ** 2, axis=1, keepdims=True)
    h = (h - mu) / jnp.sqrt(var + 1e-5) * g[...] + b[...]
    hout[...] = jnp.maximum(h, 0.0)


def _dot3(a, b):
    ah = a.astype(jnp.bfloat16)
    al = (a - ah.astype(_f32)).astype(jnp.bfloat16)
    bh = b.astype(jnp.bfloat16)
    bl = (b - bh.astype(_f32)).astype(jnp.bfloat16)

    def d(x, y):
        return jax.lax.dot_general(x, y, (((1,), (0,)), ((), ())),
                                   preferred_element_type=_f32)

    return d(ah, bh) + d(ah, bl) + d(al, bh)


def _edge23_body(gg, eav, wcat, we, bb, mout):
    g = gg[...]
    ea = eav[...]
    a = _dot3(g, wcat[...]) + _dot3(ea, we[...]) + bb[...]
    m = _sigmoid(a[:, 0:64]) * _softplus(a[:, 64:128])
    mout[...] = jnp.concatenate([m, jnp.zeros_like(m)], axis=1)


def _act_body(a, hout):
    hout[...] = jnp.maximum(jnp.clip(a[...], -1e6, 1e6), 0.0)


def _poolact_body(a, bt, out):
    i = pl.program_id(0)
    h = jnp.maximum(jnp.clip(a[...], -1e6, 1e6), 0.0)
    haug = jnp.concatenate([h, jnp.ones((h.shape[0], 8), _f32)], axis=1)
    gids = bt[...].reshape(h.shape[0], 1)
    iota = jax.lax.broadcasted_iota(jnp.int32, (h.shape[0], NUM_GRAPHS), 1)
    onehot = (gids == iota).astype(_f32)
    contrib = jax.lax.dot_general(
        onehot, haug, (((0,), (0,)), ((), ())),
        preferred_element_type=_f32, precision=_PREC)

    @pl.when(i == 0)
    def _():
        out[...] = jnp.zeros_like(out)

    out[...] += contrib


def _head_body(pa, fw, fb, g, b, hw, hb, out):
    p = pa[...]
    pooled = p[:, 0:64] / jnp.maximum(p[:, 64:65], 1.0)
    gg = _dot(pooled, fw[...]) + fb[...]
    mu = jnp.mean(gg, axis=1, keepdims=True)
    var = jnp.mean((gg - mu) ** 2, axis=1, keepdims=True)
    gg = (gg - mu) / jnp.sqrt(var + 1e-5) * g[...] + b[...]
    gg = jnp.clip(jnp.maximum(gg, 0.0), -1e6, 1e6)
    out[...] = _dot(gg, hw[...]) + hb[...]


def _full(shape):
    return pl.BlockSpec(shape, lambda i: (0, 0))


def _nodeprep1(x8p, w1d, w1s, pw, pb):
    return pl.pallas_call(
        _nodeprep1_body,
        grid=(NG,),
        in_specs=[pl.BlockSpec((BN, 8), lambda i: (i, 0)),
                  _full((8, 16)), _full((8, 16)), _full((8, 64)),
                  _full((1, 64))],
        out_specs=[pl.BlockSpec((BN, 64), lambda i: (i, 0))] * 2,
        out_shape=[jax.ShapeDtypeStruct((NP, 64), _f32)] * 2,
    )(x8p, w1d, w1s, pw, pb)


def _edge1(gg, eav, we1, b1, pw8):
    return pl.pallas_call(
        _edge1_body,
        grid=(EG,),
        in_specs=[pl.BlockSpec((BE, 128), lambda i: (i, 0)),
                  pl.BlockSpec((BE, 16), lambda i: (i, 0)),
                  _full((16, 16)), _full((1, 16)), _full((8, 64))],
        out_specs=pl.BlockSpec((BE, 128), lambda i: (i, 0)),
        out_shape=jax.ShapeDtypeStruct((EP, 128), _f32),
    )(gg, eav, we1, b1, pw8)


def _lnact(a, g, b):
    return pl.pallas_call(
        _lnact_body,
        grid=(NG,),
        in_specs=[pl.BlockSpec((BN, 64), lambda i: (i, 0)),
                  _full((1, 64)), _full((1, 64))],
        out_specs=pl.BlockSpec((BN, 64), lambda i: (i, 0)),
        out_shape=jax.ShapeDtypeStruct((NP, 64), _f32),
    )(a, g, b)


def _edge23(gg, eav, wcat, we, bb):
    return pl.pallas_call(
        _edge23_body,
        grid=(EG,),
        in_specs=[pl.BlockSpec((BE, 128), lambda i: (i, 0)),
                  pl.BlockSpec((BE, 16), lambda i: (i, 0)),
                  _full((128, 128)), _full((16, 128)), _full((1, 128))],
        out_specs=pl.BlockSpec((BE, 128), lambda i: (i, 0)),
        out_shape=jax.ShapeDtypeStruct((EP, 128), _f32),
    )(gg, eav, wcat, we, bb)


def _act(a):
    nspec = pl.BlockSpec((BN, 64), lambda i: (i, 0))
    return pl.pallas_call(
        _act_body,
        grid=(NG,),
        in_specs=[nspec],
        out_specs=nspec,
        out_shape=jax.ShapeDtypeStruct((NP, 64), _f32),
    )(a)


def _poolact(a, bt):
    return pl.pallas_call(
        _poolact_body,
        grid=(25,),
        in_specs=[pl.BlockSpec((2000, 64), lambda i: (i, 0)),
                  pl.BlockSpec((1, 2000, 1), lambda i: (i, 0, 0))],
        out_specs=pl.BlockSpec((NUM_GRAPHS, 72), lambda i: (0, 0)),
        out_shape=jax.ShapeDtypeStruct((NUM_GRAPHS, 72), _f32),
    )(a, bt)


def _head(pa, fw, fb, g, b, hw, hb):
    return pl.pallas_call(
        _head_body,
        in_specs=[pl.BlockSpec((NUM_GRAPHS, 72), lambda: (0, 0)),
                  pl.BlockSpec((64, 64), lambda: (0, 0)),
                  pl.BlockSpec((1, 64), lambda: (0, 0)),
                  pl.BlockSpec((1, 64), lambda: (0, 0)),
                  pl.BlockSpec((1, 64), lambda: (0, 0)),
                  pl.BlockSpec((64, 8), lambda: (0, 0)),
                  pl.BlockSpec((1, 8), lambda: (0, 0))],
        out_specs=pl.BlockSpec((NUM_GRAPHS, 8), lambda: (0, 0)),
        out_shape=jax.ShapeDtypeStruct((NUM_GRAPHS, 8), _f32),
    )(pa, fw, fb, g, b, hw, hb)


def kernel(x, edge_index, edge_attr, batch, c1_Wf, c1_bf, c1_Ws, c1_bs,
           proj_W, proj_b, c2_Wf, c2_bf, c2_Ws, c2_bs, c3_Wf, c3_bf,
           c3_Ws, c3_bs, fc1_W, fc1_b, ln_g, ln_b, head_W, head_b):
    f32 = _f32
    x8p = jnp.zeros((NP, 8), f32).at[:N, :3].set(x)
    dst = edge_index[1].astype(jnp.int32)
    src = edge_index[0].astype(jnp.int32)
    padi = jnp.full((EP - E,), PAD_NODE, jnp.int32)
    dst2 = jnp.concatenate([dst, padi]).reshape(NR, 128)
    src2 = jnp.concatenate([src, padi]).reshape(NR, 128)
    eav = jnp.zeros((EP, 16), f32).at[:E, :].set(edge_attr)

    w1d = jnp.zeros((8, 16), f32).at[0:3, 0:3].set(c1_Wf[0:3]) \
                                 .at[0:3, 8:11].set(c1_Ws[0:3])
    w1s = jnp.zeros((8, 16), f32).at[0:3, 0:3].set(c1_Wf[3:6]) \
                                 .at[0:3, 8:11].set(c1_Ws[3:6])
    we1 = jnp.zeros((16, 16), f32).at[:, 0:3].set(c1_Wf[6:22]) \
                                  .at[:, 8:11].set(c1_Ws[6:22])
    b1 = jnp.zeros((1, 16), f32).at[0, 0:3].set(c1_bf).at[0, 8:11].set(c1_bs)

    pw8 = jnp.zeros((8, 64), f32).at[0:3, :].set(proj_W)
    pb = proj_b.reshape(1, 64)
    lg = ln_g.reshape(1, 64)
    lb = ln_b.reshape(1, 64)

    def wsplit(wf, wsm, bf, bs):
        wcat = jnp.concatenate(
            [jnp.concatenate([wf[0:64], wsm[0:64]], axis=1),
             jnp.concatenate([wf[64:128], wsm[64:128]], axis=1)], axis=0)
        we = jnp.concatenate([wf[128:144], wsm[128:144]], axis=1)
        bb = jnp.concatenate([bf, bs]).reshape(1, 128)
        return wcat, we, bb

    wcat2, we2, bb2 = wsplit(c2_Wf, c2_Ws, c2_bf, c2_bs)
    wcat3, we3, bb3 = wsplit(c3_Wf, c3_Ws, c3_bf, c3_bs)

    bt = batch.astype(jnp.int32).reshape(25, 2000, 1)
    fw = fc1_W
    fb = fc1_b.reshape(1, 64)
    hw8 = jnp.zeros((64, 8), f32).at[:, 0:5].set(head_W)
    hb8 = jnp.zeros((1, 8), f32).at[0, 0:5].set(head_b)

    t1, xp = _nodeprep1(x8p, w1d, w1s, pw8, pb)
    g1 = _sc_gather23(t1, dst2, src2)
    m1 = _edge1(g1, eav, we1, b1, pw8)
    a1 = _sc_scatter23(m1, dst2, xp)
    h = _lnact(a1, lg, lb)

    gg = _sc_gather23(h, dst2, src2)
    m2 = _edge23(gg, eav, wcat2, we2, bb2)
    a2 = _sc_scatter23(m2, dst2, h)
    h2 = _act(a2)

    gg = _sc_gather23(h2, dst2, src2)
    m3 = _edge23(gg, eav, wcat3, we3, bb3)
    a3 = _sc_scatter23(m3, dst2, h2)

    pa = _poolact(a3, bt)
    out8 = _head(pa, fw, fb, lg, lb, hw8, hb8)
    return out8[:, 0:5]


# speedup vs baseline: 1.1515x; 1.1515x over previous
import functools

import jax
import jax.numpy as jnp
from jax import lax
from jax.experimental import pallas as pl
from jax.experimental.pallas import tpu as pltpu
from jax.experimental.pallas import tpu_sc as plsc

N = 50000
E = 800000
HID = 64
NUM_GRAPHS = 512

NP = 51200
EP = 802816
NR = EP // 128
PAD_NODE = N

NW = 32
EH = EP // 2
NRH = NR // 2
RWH = NRH // NW
BE = 2048
EGH = EH // BE
BN = 2048
NG = NP // BN

_mesh = plsc.VectorSubcoreMesh(core_axis_name="c", subcore_axis_name="s")
_f32 = jnp.float32
_SCPARAMS = pltpu.CompilerParams(use_tc_tiling_on_sc=False)


_KR = 2
_NCH = RWH // _KR


@functools.partial(
    pl.kernel,
    out_type=jax.ShapeDtypeStruct((EH, 128), _f32),
    mesh=_mesh,
    compiler_params=_SCPARAMS,
    scratch_types=[
        pltpu.VMEM((_KR, 128), jnp.int32),
        pltpu.VMEM((_KR, 128), jnp.int32),
        pltpu.VMEM((_KR, 128), jnp.int32),
        pltpu.VMEM((_KR, 128), jnp.int32),
        pltpu.VMEM((_KR * 128, 64), _f32),
        pltpu.VMEM((_KR * 128, 64), _f32),
        pltpu.VMEM((_KR * 128, 64), _f32),
        pltpu.VMEM((_KR * 128, 64), _f32),
        pltpu.SemaphoreType.DMA,
        pltpu.SemaphoreType.DMA,
    ],
)
def _sc_gather23(h, dst2, src2, out,
                 ia0, is0, ia1, is1, bd0, bs0, bd1, bs1, sem0, sem1):
    c = lax.axis_index("c")
    s = lax.axis_index("s")
    w = s * 2 + c
    wbase = w * RWH

    def load_and_fire(chunk, ia, is_, bd, bs, sem):
        rbase = wbase + chunk * _KR
        pltpu.sync_copy(dst2.at[pl.ds(rbase, _KR), :], ia)
        pltpu.sync_copy(src2.at[pl.ds(rbase, _KR), :], is_)
        for j in range(_KR):
            sl = pl.ds(j * 128, 128)
            pltpu.async_copy(h.at[ia.at[j]], bd.at[sl, :], sem)
            pltpu.async_copy(h.at[is_.at[j]], bs.at[sl, :], sem)

    def drain_and_write(chunk, ia, is_, bd, bs, sem):
        for j in range(_KR):
            sl = pl.ds(j * 128, 128)
            pltpu.make_async_copy(h.at[ia.at[j]], bd.at[sl, :], sem).wait()
            pltpu.make_async_copy(h.at[is_.at[j]], bs.at[sl, :], sem).wait()
        ebase = (wbase + chunk * _KR) * 128
        pltpu.sync_copy(bd, out.at[pl.ds(ebase, _KR * 128), pl.ds(0, 64)])
        pltpu.sync_copy(bs, out.at[pl.ds(ebase, _KR * 128), pl.ds(64, 64)])

    load_and_fire(0, ia0, is0, bd0, bs0, sem0)

    def body(g, _):
        c0 = 2 * g
        load_and_fire(c0 + 1, ia1, is1, bd1, bs1, sem1)
        drain_and_write(c0, ia0, is0, bd0, bs0, sem0)

        @pl.when(c0 + 2 < _NCH)
        def _():
            load_and_fire(c0 + 2, ia0, is0, bd0, bs0, sem0)

        drain_and_write(c0 + 1, ia1, is1, bd1, bs1, sem1)
        return 0

    lax.fori_loop(0, _NCH // 2, body, 0)
    drain_and_write(_NCH - 1, ia0, is0, bd0, bs0, sem0)


_SKR = 7
_RPT = NRH // 16
_SNCH = _RPT // _SKR


@functools.partial(
    pl.kernel,
    out_type=jax.ShapeDtypeStruct((NP, 64), _f32),
    mesh=_mesh,
    compiler_params=_SCPARAMS,
    scratch_types=[
        pltpu.VMEM((_SKR, 128), jnp.int32),
        pltpu.VMEM((_SKR, 128), jnp.int32),
        pltpu.VMEM((_SKR * 128, 16), _f32),
        pltpu.VMEM((_SKR * 128, 16), _f32),
        pltpu.VMEM_SHARED((NP, 16), _f32),
        pltpu.SemaphoreType.DMA,
        pltpu.SemaphoreType.DMA,
        pltpu.SemaphoreType.DMA,
    ],
)
def _sc_scatter23(m, dst2, h, accout,
                  ix0, ix1, mb0, mb1, acc, sem0, sem1, lsem):
    c = lax.axis_index("c")
    s = lax.axis_index("s")
    rows = pl.ds(s * 3200, 3200)

    for q in range(2):
        cols = pl.ds(c * 32 + q * 16, 16)

        pltpu.sync_copy(h.at[rows, cols], acc.at[rows, :])
        plsc.subcore_barrier()

        def load(chunk, ix, mb, cols=cols):
            rbase = s * _RPT + chunk * _SKR
            pltpu.sync_copy(dst2.at[pl.ds(rbase, _SKR), :], ix)
            return pltpu.async_copy(
                m.at[pl.ds(rbase * 128, _SKR * 128), cols], mb, lsem)

        def scat(ix, mb, sem):
            for j in range(_SKR):
                pltpu.async_copy(mb.at[pl.ds(j * 128, 128), :],
                                 acc.at[ix.at[j]], sem, add=True)

        def drain(ix, mb, sem):
            for j in range(_SKR):
                pltpu.make_async_copy(mb.at[pl.ds(j * 128, 128), :],
                                      acc.at[ix.at[j]], sem).wait()

        load(0, ix0, mb0).wait()

        def body(g, _):
            c0 = 2 * g
            ld = load(c0 + 1, ix1, mb1)
            scat(ix0, mb0, sem0)
            ld.wait()
            drain(ix0, mb0, sem0)

            @pl.when(c0 + 2 < _SNCH)
            def _():
                load(c0 + 2, ix0, mb0).wait()

            scat(ix1, mb1, sem1)
            drain(ix1, mb1, sem1)
            return 0

        lax.fori_loop(0, _SNCH // 2, body, 0)
        plsc.subcore_barrier()
        pltpu.sync_copy(acc.at[rows, :], accout.at[rows, cols])
        plsc.subcore_barrier()


_PREC = jax.lax.Precision.HIGHEST


def _dot(a, b, prec=_PREC):
    return jnp.dot(a, b, preferred_element_type=_f32, precision=prec)


def _sigmoid(x):
    return 1.0 / (1.0 + jnp.exp(-x))


def _softplus(x):
    return jnp.maximum(x, 0.0) + jnp.log1p(jnp.exp(-jnp.abs(x)))


def _nodeprep1_body(x8, w1d, w1s, pw, pb, t1, xp):
    xv = x8[...]
    z = jnp.zeros((xv.shape[0], 32), _f32)
    t1[...] = jnp.concatenate([_dot(xv, w1d[...]), _dot(xv, w1s[...]), z],
                              axis=1)
    xp[...] = _dot(xv, pw[...]) + pb[...]


def _edge1_body(gg, eav, we1, b1, pw8, mout):
    g = gg[...]
    ea = eav[...]
    a = g[:, 0:16] + g[:, 80:96] + _dot3(ea, we1[...]) + b1[...]
    m = _sigmoid(a[:, 0:8]) * _softplus(a[:, 8:16])
    m1 = _dot(m, pw8[...])
    mout[...] = jnp.concatenate([m1, jnp.zeros_like(m1)], axis=1)


def _lnact_body(a, g, b, hout):
    h = a[...]
    mu = jnp.mean(h, axis=1, keepdims=True)
    var = jnp.mean((h - mu) ** 2, axis=1, keepdims=True)
    h = (h - mu) / jnp.sqrt(var + 1e-5) * g[...] + b[...]
    hout[...] = jnp.maximum(h, 0.0)


def _dot3(a, b):
    ah = a.astype(jnp.bfloat16)
    al = (a - ah.astype(_f32)).astype(jnp.bfloat16)
    bh = b.astype(jnp.bfloat16)
    bl = (b - bh.astype(_f32)).astype(jnp.bfloat16)

    def d(x, y):
        return jax.lax.dot_general(x, y, (((1,), (0,)), ((), ())),
                                   preferred_element_type=_f32)

    return d(ah, bh) + d(ah, bl) + d(al, bh)


def _edge23_body(gg, eav, wcat, we, bb, mout):
    g = gg[...]
    ea = eav[...]
    a = _dot3(g, wcat[...]) + _dot3(ea, we[...]) + bb[...]
    m = _sigmoid(a[:, 0:64]) * _softplus(a[:, 64:128])
    mout[...] = jnp.concatenate([m, jnp.zeros_like(m)], axis=1)


def _act_body(a, hout):
    hout[...] = jnp.maximum(jnp.clip(a[...], -1e6, 1e6), 0.0)


def _poolact_body(a, bt, out):
    i = pl.program_id(0)
    h = jnp.maximum(jnp.clip(a[...], -1e6, 1e6), 0.0)
    haug = jnp.concatenate([h, jnp.ones((h.shape[0], 8), _f32)], axis=1)
    gids = bt[...].reshape(h.shape[0], 1)
    iota = jax.lax.broadcasted_iota(jnp.int32, (h.shape[0], NUM_GRAPHS), 1)
    onehot = (gids == iota).astype(_f32)
    contrib = jax.lax.dot_general(
        onehot, haug, (((0,), (0,)), ((), ())),
        preferred_element_type=_f32, precision=_PREC)

    @pl.when(i == 0)
    def _():
        out[...] = jnp.zeros_like(out)

    out[...] += contrib


def _head_body(pa, fw, fb, g, b, hw, hb, out):
    p = pa[...]
    pooled = p[:, 0:64] / jnp.maximum(p[:, 64:65], 1.0)
    gg = _dot(pooled, fw[...]) + fb[...]
    mu = jnp.mean(gg, axis=1, keepdims=True)
    var = jnp.mean((gg - mu) ** 2, axis=1, keepdims=True)
    gg = (gg - mu) / jnp.sqrt(var + 1e-5) * g[...] + b[...]
    gg = jnp.clip(jnp.maximum(gg, 0.0), -1e6, 1e6)
    out[...] = _dot(gg, hw[...]) + hb[...]


def _full(shape):
    return pl.BlockSpec(shape, lambda i: (0, 0))


def _nodeprep1(x8p, w1d, w1s, pw, pb):
    return pl.pallas_call(
        _nodeprep1_body,
        grid=(NG,),
        in_specs=[pl.BlockSpec((BN, 8), lambda i: (i, 0)),
                  _full((8, 16)), _full((8, 16)), _full((8, 64)),
                  _full((1, 64))],
        out_specs=[pl.BlockSpec((BN, 64), lambda i: (i, 0))] * 2,
        out_shape=[jax.ShapeDtypeStruct((NP, 64), _f32)] * 2,
    )(x8p, w1d, w1s, pw, pb)


def _edge1(gg, eav, we1, b1, pw8):
    return pl.pallas_call(
        _edge1_body,
        grid=(EGH,),
        in_specs=[pl.BlockSpec((BE, 128), lambda i: (i, 0)),
                  pl.BlockSpec((BE, 16), lambda i: (i, 0)),
                  _full((16, 16)), _full((1, 16)), _full((8, 64))],
        out_specs=pl.BlockSpec((BE, 128), lambda i: (i, 0)),
        out_shape=jax.ShapeDtypeStruct((EH, 128), _f32),
    )(gg, eav, we1, b1, pw8)


def _lnact(a, g, b):
    return pl.pallas_call(
        _lnact_body,
        grid=(NG,),
        in_specs=[pl.BlockSpec((BN, 64), lambda i: (i, 0)),
                  _full((1, 64)), _full((1, 64))],
        out_specs=pl.BlockSpec((BN, 64), lambda i: (i, 0)),
        out_shape=jax.ShapeDtypeStruct((NP, 64), _f32),
    )(a, g, b)


def _edge23(gg, eav, wcat, we, bb):
    return pl.pallas_call(
        _edge23_body,
        grid=(EGH,),
        in_specs=[pl.BlockSpec((BE, 128), lambda i: (i, 0)),
                  pl.BlockSpec((BE, 16), lambda i: (i, 0)),
                  _full((128, 128)), _full((16, 128)), _full((1, 128))],
        out_specs=pl.BlockSpec((BE, 128), lambda i: (i, 0)),
        out_shape=jax.ShapeDtypeStruct((EH, 128), _f32),
    )(gg, eav, wcat, we, bb)


def _act(a):
    nspec = pl.BlockSpec((BN, 64), lambda i: (i, 0))
    return pl.pallas_call(
        _act_body,
        grid=(NG,),
        in_specs=[nspec],
        out_specs=nspec,
        out_shape=jax.ShapeDtypeStruct((NP, 64), _f32),
    )(a)


def _poolact(a, bt):
    return pl.pallas_call(
        _poolact_body,
        grid=(25,),
        in_specs=[pl.BlockSpec((2000, 64), lambda i: (i, 0)),
                  pl.BlockSpec((1, 2000, 1), lambda i: (i, 0, 0))],
        out_specs=pl.BlockSpec((NUM_GRAPHS, 72), lambda i: (0, 0)),
        out_shape=jax.ShapeDtypeStruct((NUM_GRAPHS, 72), _f32),
    )(a, bt)


def _head(pa, fw, fb, g, b, hw, hb):
    return pl.pallas_call(
        _head_body,
        in_specs=[pl.BlockSpec((NUM_GRAPHS, 72), lambda: (0, 0)),
                  pl.BlockSpec((64, 64), lambda: (0, 0)),
                  pl.BlockSpec((1, 64), lambda: (0, 0)),
                  pl.BlockSpec((1, 64), lambda: (0, 0)),
                  pl.BlockSpec((1, 64), lambda: (0, 0)),
                  pl.BlockSpec((64, 8), lambda: (0, 0)),
                  pl.BlockSpec((1, 8), lambda: (0, 0))],
        out_specs=pl.BlockSpec((NUM_GRAPHS, 8), lambda: (0, 0)),
        out_shape=jax.ShapeDtypeStruct((NUM_GRAPHS, 8), _f32),
    )(pa, fw, fb, g, b, hw, hb)


def kernel(x, edge_index, edge_attr, batch, c1_Wf, c1_bf, c1_Ws, c1_bs,
           proj_W, proj_b, c2_Wf, c2_bf, c2_Ws, c2_bs, c3_Wf, c3_bf,
           c3_Ws, c3_bs, fc1_W, fc1_b, ln_g, ln_b, head_W, head_b):
    f32 = _f32
    x8p = jnp.zeros((NP, 8), f32).at[:N, :3].set(x)
    dst = edge_index[1].astype(jnp.int32)
    src = edge_index[0].astype(jnp.int32)
    padi = jnp.full((EP - E,), PAD_NODE, jnp.int32)
    dst2 = jnp.concatenate([dst, padi]).reshape(NR, 128)
    src2 = jnp.concatenate([src, padi]).reshape(NR, 128)
    d2a, d2b = dst2[:NRH], dst2[NRH:]
    s2a, s2b = src2[:NRH], src2[NRH:]
    eap = jnp.zeros((EP, 16), f32).at[:E, :].set(edge_attr)
    eava, eavb = eap[:EH], eap[EH:]

    w1d = jnp.zeros((8, 16), f32).at[0:3, 0:3].set(c1_Wf[0:3]) \
                                 .at[0:3, 8:11].set(c1_Ws[0:3])
    w1s = jnp.zeros((8, 16), f32).at[0:3, 0:3].set(c1_Wf[3:6]) \
                                 .at[0:3, 8:11].set(c1_Ws[3:6])
    we1 = jnp.zeros((16, 16), f32).at[:, 0:3].set(c1_Wf[6:22]) \
                                  .at[:, 8:11].set(c1_Ws[6:22])
    b1 = jnp.zeros((1, 16), f32).at[0, 0:3].set(c1_bf).at[0, 8:11].set(c1_bs)

    pw8 = jnp.zeros((8, 64), f32).at[0:3, :].set(proj_W)
    pb = proj_b.reshape(1, 64)
    lg = ln_g.reshape(1, 64)
    lb = ln_b.reshape(1, 64)

    def wsplit(wf, wsm, bf, bs):
        wcat = jnp.concatenate(
            [jnp.concatenate([wf[0:64], wsm[0:64]], axis=1),
             jnp.concatenate([wf[64:128], wsm[64:128]], axis=1)], axis=0)
        we = jnp.concatenate([wf[128:144], wsm[128:144]], axis=1)
        bb = jnp.concatenate([bf, bs]).reshape(1, 128)
        return wcat, we, bb

    wcat2, we2, bb2 = wsplit(c2_Wf, c2_Ws, c2_bf, c2_bs)
    wcat3, we3, bb3 = wsplit(c3_Wf, c3_Ws, c3_bf, c3_bs)

    bt = batch.astype(jnp.int32).reshape(25, 2000, 1)
    fw = fc1_W
    fb = fc1_b.reshape(1, 64)
    hw8 = jnp.zeros((64, 8), f32).at[:, 0:5].set(head_W)
    hb8 = jnp.zeros((1, 8), f32).at[0, 0:5].set(head_b)

    t1, xp = _nodeprep1(x8p, w1d, w1s, pw8, pb)
    ga = _sc_gather23(t1, d2a, s2a)
    gb = _sc_gather23(t1, d2b, s2b)
    ma = _edge1(ga, eava, we1, b1, pw8)
    mb = _edge1(gb, eavb, we1, b1, pw8)
    aa = _sc_scatter23(ma, d2a, xp)
    ab = _sc_scatter23(mb, d2b, aa)
    h = _lnact(ab, lg, lb)

    ga = _sc_gather23(h, d2a, s2a)
    gb = _sc_gather23(h, d2b, s2b)
    ma = _edge23(ga, eava, wcat2, we2, bb2)
    mb = _edge23(gb, eavb, wcat2, we2, bb2)
    aa = _sc_scatter23(ma, d2a, h)
    ab = _sc_scatter23(mb, d2b, aa)
    h2 = _act(ab)

    ga = _sc_gather23(h2, d2a, s2a)
    gb = _sc_gather23(h2, d2b, s2b)
    ma = _edge23(ga, eava, wcat3, we3, bb3)
    mb = _edge23(gb, eavb, wcat3, we3, bb3)
    aa = _sc_scatter23(ma, d2a, h2)
    a3 = _sc_scatter23(mb, d2b, aa)

    pa = _poolact(a3, bt)
    out8 = _head(pa, fw, fb, lg, lb, hw8, hb8)
    return out8[:, 0:5]


# speedup vs baseline: 1.2643x; 1.0979x over previous
import functools

import jax
import jax.numpy as jnp
from jax import lax
from jax.experimental import pallas as pl
from jax.experimental.pallas import tpu as pltpu
from jax.experimental.pallas import tpu_sc as plsc

N = 50000
E = 800000
HID = 64
NUM_GRAPHS = 512

NP = 51200
EP = 802816
NR = EP // 128
PAD_NODE = N

NW = 32
EH = EP // 2
NRH = NR // 2
RWH = NRH // NW
BE = 2048
EGH = EH // BE
BN = 2048
NG = NP // BN

_mesh = plsc.VectorSubcoreMesh(core_axis_name="c", subcore_axis_name="s")
_f32 = jnp.float32
_SCPARAMS = pltpu.CompilerParams(use_tc_tiling_on_sc=False)


_KR = 2
_NCH = RWH // _KR


@functools.partial(
    pl.kernel,
    out_type=jax.ShapeDtypeStruct((EH, 128), _f32),
    mesh=_mesh,
    compiler_params=_SCPARAMS,
    scratch_types=[
        pltpu.VMEM((_KR, 128), jnp.int32),
        pltpu.VMEM((_KR, 128), jnp.int32),
        pltpu.VMEM((_KR, 128), jnp.int32),
        pltpu.VMEM((_KR, 128), jnp.int32),
        pltpu.VMEM((_KR * 128, 64), _f32),
        pltpu.VMEM((_KR * 128, 64), _f32),
        pltpu.VMEM((_KR * 128, 64), _f32),
        pltpu.VMEM((_KR * 128, 64), _f32),
        pltpu.SemaphoreType.DMA,
        pltpu.SemaphoreType.DMA,
    ],
)
def _sc_gather23(h, dst2, src2, out,
                 ia0, is0, ia1, is1, bd0, bs0, bd1, bs1, sem0, sem1):
    c = lax.axis_index("c")
    s = lax.axis_index("s")
    w = s * 2 + c
    wbase = w * RWH

    def load_and_fire(chunk, ia, is_, bd, bs, sem):
        rbase = wbase + chunk * _KR
        pltpu.sync_copy(dst2.at[pl.ds(rbase, _KR), :], ia)
        pltpu.sync_copy(src2.at[pl.ds(rbase, _KR), :], is_)
        for j in range(_KR):
            sl = pl.ds(j * 128, 128)
            pltpu.async_copy(h.at[ia.at[j]], bd.at[sl, :], sem)
            pltpu.async_copy(h.at[is_.at[j]], bs.at[sl, :], sem)

    def drain_and_write(chunk, ia, is_, bd, bs, sem):
        for j in range(_KR):
            sl = pl.ds(j * 128, 128)
            pltpu.make_async_copy(h.at[ia.at[j]], bd.at[sl, :], sem).wait()
            pltpu.make_async_copy(h.at[is_.at[j]], bs.at[sl, :], sem).wait()
        ebase = (wbase + chunk * _KR) * 128
        pltpu.sync_copy(bd, out.at[pl.ds(ebase, _KR * 128), pl.ds(0, 64)])
        pltpu.sync_copy(bs, out.at[pl.ds(ebase, _KR * 128), pl.ds(64, 64)])

    load_and_fire(0, ia0, is0, bd0, bs0, sem0)

    def body(g, _):
        c0 = 2 * g
        load_and_fire(c0 + 1, ia1, is1, bd1, bs1, sem1)
        drain_and_write(c0, ia0, is0, bd0, bs0, sem0)

        @pl.when(c0 + 2 < _NCH)
        def _():
            load_and_fire(c0 + 2, ia0, is0, bd0, bs0, sem0)

        drain_and_write(c0 + 1, ia1, is1, bd1, bs1, sem1)
        return 0

    lax.fori_loop(0, _NCH // 2, body, 0)
    drain_and_write(_NCH - 1, ia0, is0, bd0, bs0, sem0)


_SKR = 7
_RPT = NRH // 16
_SNCH = _RPT // _SKR


@functools.partial(
    pl.kernel,
    out_type=jax.ShapeDtypeStruct((NP, 64), _f32),
    mesh=_mesh,
    compiler_params=_SCPARAMS,
    scratch_types=[
        pltpu.VMEM((_SKR, 128), jnp.int32),
        pltpu.VMEM((_SKR, 128), jnp.int32),
        pltpu.VMEM((_SKR * 128, 16), _f32),
        pltpu.VMEM((_SKR * 128, 16), _f32),
        pltpu.VMEM_SHARED((NP, 16), _f32),
        pltpu.SemaphoreType.DMA,
        pltpu.SemaphoreType.DMA,
        pltpu.SemaphoreType.DMA,
    ],
)
def _sc_scatter23(m, dst2, h, accout,
                  ix0, ix1, mb0, mb1, acc, sem0, sem1, lsem):
    c = lax.axis_index("c")
    s = lax.axis_index("s")
    rows = pl.ds(s * 3200, 3200)

    for q in range(2):
        cols = pl.ds(c * 32 + q * 16, 16)

        pltpu.sync_copy(h.at[rows, cols], acc.at[rows, :])
        plsc.subcore_barrier()

        def load(chunk, ix, mb, cols=cols):
            rbase = s * _RPT + chunk * _SKR
            pltpu.sync_copy(dst2.at[pl.ds(rbase, _SKR), :], ix)
            return pltpu.async_copy(
                m.at[pl.ds(rbase * 128, _SKR * 128), cols], mb, lsem)

        def scat(ix, mb, sem):
            for j in range(_SKR):
                pltpu.async_copy(mb.at[pl.ds(j * 128, 128), :],
                                 acc.at[ix.at[j]], sem, add=True)

        def drain(ix, mb, sem):
            for j in range(_SKR):
                pltpu.make_async_copy(mb.at[pl.ds(j * 128, 128), :],
                                      acc.at[ix.at[j]], sem).wait()

        load(0, ix0, mb0).wait()

        def body(g, _):
            c0 = 2 * g
            ld = load(c0 + 1, ix1, mb1)
            scat(ix0, mb0, sem0)
            ld.wait()
            drain(ix0, mb0, sem0)

            @pl.when(c0 + 2 < _SNCH)
            def _():
                load(c0 + 2, ix0, mb0).wait()

            scat(ix1, mb1, sem1)
            drain(ix1, mb1, sem1)
            return 0

        lax.fori_loop(0, _SNCH // 2, body, 0)
        plsc.subcore_barrier()
        pltpu.sync_copy(acc.at[rows, :], accout.at[rows, cols])
        plsc.subcore_barrier()


_PREC = jax.lax.Precision.HIGHEST


def _dot(a, b, prec=_PREC):
    return jnp.dot(a, b, preferred_element_type=_f32, precision=prec)


def _sigmoid(x):
    return 1.0 / (1.0 + jnp.exp(-x))


def _softplus(x):
    return jnp.maximum(x, 0.0) + jnp.log1p(jnp.exp(-jnp.abs(x)))


def _nodeprep1_body(x8, w1d, w1s, pw, pb, t1, xp):
    xv = x8[...]
    z = jnp.zeros((xv.shape[0], 32), _f32)
    t1[...] = jnp.concatenate([_dot(xv, w1d[...]), _dot(xv, w1s[...]), z],
                              axis=1)
    xp[...] = _dot(xv, pw[...]) + pb[...]


def _edge1_body(gg, eav, wbig1, b1, pw8, mout):
    g = gg[...]
    r = jax.lax.dot_general(eav[...].astype(jnp.bfloat16), wbig1[...],
                            (((1,), (0,)), ((), ())),
                            preferred_element_type=_f32)
    r = jnp.reshape(r, (g.shape[0], 128))
    a = g[:, 0:16] + g[:, 80:96] + r[:, 0:16] + b1[...]
    m = _sigmoid(a[:, 0:8]) * _softplus(a[:, 8:16])
    m1 = _dot(m, pw8[...])
    mout[...] = jnp.concatenate([m1, jnp.zeros_like(m1)], axis=1)


def _lnact_body(a, g, b, hout):
    h = a[...]
    mu = jnp.mean(h, axis=1, keepdims=True)
    var = jnp.mean((h - mu) ** 2, axis=1, keepdims=True)
    h = (h - mu) / jnp.sqrt(var + 1e-5) * g[...] + b[...]
    hout[...] = jnp.maximum(h, 0.0)


def _dot3(a, b):
    ah = a.astype(jnp.bfloat16)
    al = (a - ah.astype(_f32)).astype(jnp.bfloat16)
    bh = b.astype(jnp.bfloat16)
    bl = (b - bh.astype(_f32)).astype(jnp.bfloat16)

    def d(x, y):
        return jax.lax.dot_general(x, y, (((1,), (0,)), ((), ())),
                                   preferred_element_type=_f32)

    return d(ah, bh) + d(ah, bl) + d(al, bh)


def _edge23_body(gg, eav, wcat, wbig, bb, mout):
    g = gg[...]
    r = jax.lax.dot_general(eav[...].astype(jnp.bfloat16), wbig[...],
                            (((1,), (0,)), ((), ())),
                            preferred_element_type=_f32)
    r = jnp.reshape(r, (g.shape[0], 128))
    a = _dot3(g, wcat[...]) + r + bb[...]
    m = _sigmoid(a[:, 0:64]) * _softplus(a[:, 64:128])
    mout[...] = jnp.concatenate([m, jnp.zeros_like(m)], axis=1)


def _act_body(a, hout):
    hout[...] = jnp.maximum(jnp.clip(a[...], -1e6, 1e6), 0.0)


def _poolact_body(a, bt, out):
    i = pl.program_id(0)
    h = jnp.maximum(jnp.clip(a[...], -1e6, 1e6), 0.0)
    haug = jnp.concatenate([h, jnp.ones((h.shape[0], 8), _f32)], axis=1)
    gids = bt[...].reshape(h.shape[0], 1)
    iota = jax.lax.broadcasted_iota(jnp.int32, (h.shape[0], NUM_GRAPHS), 1)
    onehot = (gids == iota).astype(_f32)
    contrib = jax.lax.dot_general(
        onehot, haug, (((0,), (0,)), ((), ())),
        preferred_element_type=_f32, precision=_PREC)

    @pl.when(i == 0)
    def _():
        out[...] = jnp.zeros_like(out)

    out[...] += contrib


def _head_body(pa, fw, fb, g, b, hw, hb, out):
    p = pa[...]
    pooled = p[:, 0:64] / jnp.maximum(p[:, 64:65], 1.0)
    gg = _dot(pooled, fw[...]) + fb[...]
    mu = jnp.mean(gg, axis=1, keepdims=True)
    var = jnp.mean((gg - mu) ** 2, axis=1, keepdims=True)
    gg = (gg - mu) / jnp.sqrt(var + 1e-5) * g[...] + b[...]
    gg = jnp.clip(jnp.maximum(gg, 0.0), -1e6, 1e6)
    out[...] = _dot(gg, hw[...]) + hb[...]


def _full(shape):
    return pl.BlockSpec(shape, lambda i: (0, 0))


def _nodeprep1(x8p, w1d, w1s, pw, pb):
    return pl.pallas_call(
        _nodeprep1_body,
        grid=(NG,),
        in_specs=[pl.BlockSpec((BN, 8), lambda i: (i, 0)),
                  _full((8, 16)), _full((8, 16)), _full((8, 64)),
                  _full((1, 64))],
        out_specs=[pl.BlockSpec((BN, 64), lambda i: (i, 0))] * 2,
        out_shape=[jax.ShapeDtypeStruct((NP, 64), _f32)] * 2,
    )(x8p, w1d, w1s, pw, pb)


def _edge1(gg, eav, wbig1, b1, pw8):
    return pl.pallas_call(
        _edge1_body,
        grid=(EGH,),
        in_specs=[pl.BlockSpec((BE, 128), lambda i: (i, 0)),
                  pl.BlockSpec((BE // 8, 128), lambda i: (i, 0)),
                  _full((128, 1024)), _full((1, 16)), _full((8, 64))],
        out_specs=pl.BlockSpec((BE, 128), lambda i: (i, 0)),
        out_shape=jax.ShapeDtypeStruct((EH, 128), _f32),
    )(gg, eav, wbig1, b1, pw8)


def _lnact(a, g, b):
    return pl.pallas_call(
        _lnact_body,
        grid=(NG,),
        in_specs=[pl.BlockSpec((BN, 64), lambda i: (i, 0)),
                  _full((1, 64)), _full((1, 64))],
        out_specs=pl.BlockSpec((BN, 64), lambda i: (i, 0)),
        out_shape=jax.ShapeDtypeStruct((NP, 64), _f32),
    )(a, g, b)


def _edge23(gg, eav, wcat, wbig, bb):
    return pl.pallas_call(
        _edge23_body,
        grid=(EGH,),
        in_specs=[pl.BlockSpec((BE, 128), lambda i: (i, 0)),
                  pl.BlockSpec((BE // 8, 128), lambda i: (i, 0)),
                  _full((128, 128)), _full((128, 1024)), _full((1, 128))],
        out_specs=pl.BlockSpec((BE, 128), lambda i: (i, 0)),
        out_shape=jax.ShapeDtypeStruct((EH, 128), _f32),
    )(gg, eav, wcat, wbig, bb)


def _act(a):
    nspec = pl.BlockSpec((BN, 64), lambda i: (i, 0))
    return pl.pallas_call(
        _act_body,
        grid=(NG,),
        in_specs=[nspec],
        out_specs=nspec,
        out_shape=jax.ShapeDtypeStruct((NP, 64), _f32),
    )(a)


def _poolact(a, bt):
    return pl.pallas_call(
        _poolact_body,
        grid=(25,),
        in_specs=[pl.BlockSpec((2000, 64), lambda i: (i, 0)),
                  pl.BlockSpec((1, 2000, 1), lambda i: (i, 0, 0))],
        out_specs=pl.BlockSpec((NUM_GRAPHS, 72), lambda i: (0, 0)),
        out_shape=jax.ShapeDtypeStruct((NUM_GRAPHS, 72), _f32),
    )(a, bt)


def _head(pa, fw, fb, g, b, hw, hb):
    return pl.pallas_call(
        _head_body,
        in_specs=[pl.BlockSpec((NUM_GRAPHS, 72), lambda: (0, 0)),
                  pl.BlockSpec((64, 64), lambda: (0, 0)),
                  pl.BlockSpec((1, 64), lambda: (0, 0)),
                  pl.BlockSpec((1, 64), lambda: (0, 0)),
                  pl.BlockSpec((1, 64), lambda: (0, 0)),
                  pl.BlockSpec((64, 8), lambda: (0, 0)),
                  pl.BlockSpec((1, 8), lambda: (0, 0))],
        out_specs=pl.BlockSpec((NUM_GRAPHS, 8), lambda: (0, 0)),
        out_shape=jax.ShapeDtypeStruct((NUM_GRAPHS, 8), _f32),
    )(pa, fw, fb, g, b, hw, hb)


def kernel(x, edge_index, edge_attr, batch, c1_Wf, c1_bf, c1_Ws, c1_bs,
           proj_W, proj_b, c2_Wf, c2_bf, c2_Ws, c2_bs, c3_Wf, c3_bf,
           c3_Ws, c3_bs, fc1_W, fc1_b, ln_g, ln_b, head_W, head_b):
    f32 = _f32
    x8p = jnp.zeros((NP, 8), f32).at[:N, :3].set(x)
    dst = edge_index[1].astype(jnp.int32)
    src = edge_index[0].astype(jnp.int32)
    padi = jnp.full((EP - E,), PAD_NODE, jnp.int32)
    dst2 = jnp.concatenate([dst, padi]).reshape(NR, 128)
    src2 = jnp.concatenate([src, padi]).reshape(NR, 128)
    d2a, d2b = dst2[:NRH], dst2[NRH:]
    s2a, s2b = src2[:NRH], src2[NRH:]
    eav = jnp.zeros((EP // 8, 128), f32).at[:E // 8, :].set(
        edge_attr.reshape(E // 8, 128))
    eava, eavb = eav[:EH // 8], eav[EH // 8:]

    w1d = jnp.zeros((8, 16), f32).at[0:3, 0:3].set(c1_Wf[0:3]) \
                                 .at[0:3, 8:11].set(c1_Ws[0:3])
    w1s = jnp.zeros((8, 16), f32).at[0:3, 0:3].set(c1_Wf[3:6]) \
                                 .at[0:3, 8:11].set(c1_Ws[3:6])
    we1 = jnp.zeros((16, 128), f32).at[:, 0:3].set(c1_Wf[6:22]) \
                                   .at[:, 8:11].set(c1_Ws[6:22])
    wbig1 = (jnp.eye(8, dtype=f32)[:, None, :, None]
             * we1.astype(jnp.bfloat16).astype(f32)[None, :, None, :]) \
        .reshape(128, 1024).astype(jnp.bfloat16)
    b1 = jnp.zeros((1, 16), f32).at[0, 0:3].set(c1_bf).at[0, 8:11].set(c1_bs)

    pw8 = jnp.zeros((8, 64), f32).at[0:3, :].set(proj_W)
    pb = proj_b.reshape(1, 64)
    lg = ln_g.reshape(1, 64)
    lb = ln_b.reshape(1, 64)

    def wsplit(wf, wsm, bf, bs):
        wcat = jnp.concatenate(
            [jnp.concatenate([wf[0:64], wsm[0:64]], axis=1),
             jnp.concatenate([wf[64:128], wsm[64:128]], axis=1)], axis=0)
        we = jnp.concatenate([wf[128:144], wsm[128:144]], axis=1)
        wbig = (jnp.eye(8, dtype=_f32)[:, None, :, None]
                * we.astype(jnp.bfloat16).astype(_f32)[None, :, None, :]) \
            .reshape(128, 1024).astype(jnp.bfloat16)
        bb = jnp.concatenate([bf, bs]).reshape(1, 128)
        return wcat, wbig, bb

    wcat2, we2, bb2 = wsplit(c2_Wf, c2_Ws, c2_bf, c2_bs)
    wcat3, we3, bb3 = wsplit(c3_Wf, c3_Ws, c3_bf, c3_bs)

    bt = batch.astype(jnp.int32).reshape(25, 2000, 1)
    fw = fc1_W
    fb = fc1_b.reshape(1, 64)
    hw8 = jnp.zeros((64, 8), f32).at[:, 0:5].set(head_W)
    hb8 = jnp.zeros((1, 8), f32).at[0, 0:5].set(head_b)

    t1, xp = _nodeprep1(x8p, w1d, w1s, pw8, pb)
    ga = _sc_gather23(t1, d2a, s2a)
    gb = _sc_gather23(t1, d2b, s2b)
    ma = _edge1(ga, eava, wbig1, b1, pw8)
    mb = _edge1(gb, eavb, wbig1, b1, pw8)
    aa = _sc_scatter23(ma, d2a, xp)
    ab = _sc_scatter23(mb, d2b, aa)
    h = _lnact(ab, lg, lb)

    ga = _sc_gather23(h, d2a, s2a)
    gb = _sc_gather23(h, d2b, s2b)
    ma = _edge23(ga, eava, wcat2, we2, bb2)
    mb = _edge23(gb, eavb, wcat2, we2, bb2)
    aa = _sc_scatter23(ma, d2a, h)
    ab = _sc_scatter23(mb, d2b, aa)
    h2 = _act(ab)

    ga = _sc_gather23(h2, d2a, s2a)
    gb = _sc_gather23(h2, d2b, s2b)
    ma = _edge23(ga, eava, wcat3, we3, bb3)
    mb = _edge23(gb, eavb, wcat3, we3, bb3)
    aa = _sc_scatter23(ma, d2a, h2)
    a3 = _sc_scatter23(mb, d2b, aa)

    pa = _poolact(a3, bt)
    out8 = _head(pa, fw, fb, lg, lb, hw8, hb8)
    return out8[:, 0:5]
